# baseline (speedup 1.0000x reference)
import functools

import jax
import jax.numpy as jnp
from jax import lax
from jax.experimental import pallas as pl
from jax.experimental.pallas import tpu as pltpu
from jax.experimental.pallas import tpu_sc as plsc

MAXLEN_ = 200
EMBED_ = 64
BATCH_ = 1024
NWORK_ = 32
BBLK_ = 128
NBBLK_ = BATCH_ // BBLK_
NTASK_ = MAXLEN_ * NBBLK_
TPW_ = NTASK_ // NWORK_


def _emb_kernel(xt_hbm, tok_hbm, pos_hbm, out_hbm, pos_v, idx_v,
                tok0, tok1, tr0, tr1, gs0, gs1, os0, os1):
    nc = 2
    wid = lax.axis_index("s") * nc + lax.axis_index("c")
    t0 = wid * TPW_

    pltpu.sync_copy(pos_hbm, pos_v)
    pltpu.sync_copy(xt_hbm.at[pl.ds(t0, TPW_)], idx_v)

    toks = (tok0, tok1)
    trs = (tr0, tr1)
    gsems = (gs0, gs1)
    osems = (os0, os1)

    rows_c = [jb * 16 + lax.iota(jnp.int32, 16) for jb in range(BBLK_ // 16)]

    def start_gather(i, b):
        pltpu.async_copy(tok_hbm.at[idx_v.at[i]], toks[b], gsems[b])

    def wait_gather(b):
        pltpu.make_async_copy(tok_hbm.at[idx_v.at[0]], toks[b], gsems[b]).wait()

    def start_out(t, b):
        s = t // NBBLK_
        tj = lax.rem(t, NBBLK_)
        pltpu.async_copy(trs[b], out_hbm.at[s, slice(None), tj], osems[b])

    def wait_out(b):
        pltpu.make_async_copy(trs[b], out_hbm.at[0, slice(None), 0],
                              osems[b]).wait()

    start_gather(0, 0)

    def super_body(kk, carry):
        for b in (0, 1):
            i = kk * 2 + b
            t = t0 + i
            nb = 1 - b
            wait_gather(b)

            @pl.when(i + 1 < TPW_)
            def _():
                start_gather(i + 1, nb)

            @pl.when(i >= 2)
            def _():
                wait_out(b)

            s = t // NBBLK_
            svec = jnp.broadcast_to(s, (16,))
            tok = toks[b]
            tr = trs[b]

            @plsc.parallel_loop(0, EMBED_, unroll=8)
            def e_loop(e):
                ti = e >> 3
                r = e & 7
                evec = jnp.broadcast_to(e, (16,))
                pvec = plsc.load_gather(pos_v, [svec, evec])
                for jb in range(BBLK_ // 16):
                    vals = plsc.load_gather(tok, [rows_c[jb], evec])
                    tr[ti, r, pl.ds(jb * 16, 16)] = vals + pvec

            start_out(t, b)
        return carry

    lax.fori_loop(0, TPW_ // 2, super_body, 0)
    wait_out(0)
    wait_out(1)


def kernel(x, token_table, pos_table):
    batch, seqlen = x.shape
    xt = jnp.transpose(x.astype(jnp.int32), (1, 0)).reshape(NTASK_, BBLK_)

    mesh = plsc.VectorSubcoreMesh(core_axis_name="c", subcore_axis_name="s")
    run = functools.partial(
        pl.kernel,
        mesh=mesh,
        compiler_params=pltpu.CompilerParams(
            use_tc_tiling_on_sc=False, needs_layout_passes=False),
        out_type=jax.ShapeDtypeStruct(
            (MAXLEN_, EMBED_ // 8, NBBLK_, 8, BBLK_), jnp.float32),
        scratch_types=[
            pltpu.VMEM((MAXLEN_, EMBED_), jnp.float32),
            pltpu.VMEM((TPW_, BBLK_), jnp.int32),
            pltpu.VMEM((BBLK_, EMBED_), jnp.float32),
            pltpu.VMEM((BBLK_, EMBED_), jnp.float32),
            pltpu.VMEM((EMBED_ // 8, 8, BBLK_), jnp.float32),
            pltpu.VMEM((EMBED_ // 8, 8, BBLK_), jnp.float32),
            pltpu.SemaphoreType.DMA,
            pltpu.SemaphoreType.DMA,
            pltpu.SemaphoreType.DMA,
            pltpu.SemaphoreType.DMA,
        ],
    )(_emb_kernel)
    out5 = run(xt, token_table, pos_table)
    out = jnp.transpose(out5, (0, 1, 3, 2, 4)).reshape(MAXLEN_, EMBED_, BATCH_)
    return jnp.transpose(out, (2, 0, 1))


# speedup vs baseline: 1.0783x; 1.0783x over previous
import functools

import jax
import jax.numpy as jnp
from jax import lax
from jax.experimental import pallas as pl
from jax.experimental.pallas import tpu as pltpu
from jax.experimental.pallas import tpu_sc as plsc

MAXLEN_ = 200
EMBED_ = 64
BATCH_ = 1024
NWORK_ = 32
BBLK_ = 128
NBBLK_ = BATCH_ // BBLK_
NTASK_ = MAXLEN_ * NBBLK_
TPW_ = NTASK_ // NWORK_


def _emb_kernel(xt_hbm, tok_hbm, pos_hbm, out_hbm, pos_v, idx_v,
                tok0, tok1, tr0, tr1, gs0, gs1, os0, os1):
    nc = 2
    wid = lax.axis_index("s") * nc + lax.axis_index("c")
    t0 = wid * TPW_

    pltpu.sync_copy(pos_hbm, pos_v)
    pltpu.sync_copy(xt_hbm.at[pl.ds(t0, TPW_)], idx_v)

    toks = (tok0, tok1)
    trs = (tr0, tr1)
    gsems = (gs0, gs1)
    osems = (os0, os1)

    lanes = lax.iota(jnp.int32, 16)
    ti_c = [(q * 16 + lanes) >> 3 for q in range(EMBED_ // 16)]
    r_c = [(q * 16 + lanes) & 7 for q in range(EMBED_ // 16)]

    def start_gather(i, b):
        pltpu.async_copy(tok_hbm.at[idx_v.at[i]], toks[b], gsems[b])

    def wait_gather(b):
        pltpu.make_async_copy(tok_hbm.at[idx_v.at[0]], toks[b], gsems[b]).wait()

    def start_out(t, b):
        s = t // NBBLK_
        tj = lax.rem(t, NBBLK_)
        pltpu.async_copy(trs[b], out_hbm.at[s, slice(None), tj], osems[b])

    def wait_out(b):
        pltpu.make_async_copy(trs[b], out_hbm.at[0, slice(None), 0],
                              osems[b]).wait()

    start_gather(0, 0)

    def super_body(kk, carry):
        for b in (0, 1):
            i = kk * 2 + b
            t = t0 + i
            nb = 1 - b
            wait_gather(b)

            @pl.when(i + 1 < TPW_)
            def _():
                start_gather(i + 1, nb)

            @pl.when(i >= 2)
            def _():
                wait_out(b)

            s = t // NBBLK_
            tok = toks[b]
            tr = trs[b]
            prow = [pos_v[s, pl.ds(q * 16, 16)] for q in range(EMBED_ // 16)]

            @plsc.parallel_loop(0, BBLK_, unroll=4,
                                carry=jnp.broadcast_to(0, (16,)))
            def j_loop(j, jvec):
                for q in range(EMBED_ // 16):
                    vals = tok[j, pl.ds(q * 16, 16)] + prow[q]
                    plsc.store_scatter(tr, [ti_c[q], r_c[q], jvec], vals)
                return jvec + 1

            start_out(t, b)
        return carry

    lax.fori_loop(0, TPW_ // 2, super_body, 0)
    wait_out(0)
    wait_out(1)


def kernel(x, token_table, pos_table):
    batch, seqlen = x.shape
    xt = jnp.transpose(x.astype(jnp.int32), (1, 0)).reshape(NTASK_, BBLK_)

    mesh = plsc.VectorSubcoreMesh(core_axis_name="c", subcore_axis_name="s")
    run = functools.partial(
        pl.kernel,
        mesh=mesh,
        compiler_params=pltpu.CompilerParams(
            use_tc_tiling_on_sc=False, needs_layout_passes=False),
        out_type=jax.ShapeDtypeStruct(
            (MAXLEN_, EMBED_ // 8, NBBLK_, 8, BBLK_), jnp.float32),
        scratch_types=[
            pltpu.VMEM((MAXLEN_, EMBED_), jnp.float32),
            pltpu.VMEM((TPW_, BBLK_), jnp.int32),
            pltpu.VMEM((BBLK_, EMBED_), jnp.float32),
            pltpu.VMEM((BBLK_, EMBED_), jnp.float32),
            pltpu.VMEM((EMBED_ // 8, 8, BBLK_), jnp.float32),
            pltpu.VMEM((EMBED_ // 8, 8, BBLK_), jnp.float32),
            pltpu.SemaphoreType.DMA,
            pltpu.SemaphoreType.DMA,
            pltpu.SemaphoreType.DMA,
            pltpu.SemaphoreType.DMA,
        ],
    )(_emb_kernel)
    out5 = run(xt, token_table, pos_table)
    out = jnp.transpose(out5, (0, 1, 3, 2, 4)).reshape(MAXLEN_, EMBED_, BATCH_)
    return jnp.transpose(out, (2, 0, 1))


# speedup vs baseline: 1.9736x; 1.8302x over previous
import functools

import jax
import jax.numpy as jnp
from jax import lax
from jax.experimental import pallas as pl
from jax.experimental.pallas import tpu as pltpu
from jax.experimental.pallas import tpu_sc as plsc

MAXLEN_ = 200
EMBED_ = 64
BATCH_ = 1024
NWORK_ = 32
BBLK_ = 128
NBBLK_ = BATCH_ // BBLK_
NTASK_ = MAXLEN_ * NBBLK_
TPW_ = NTASK_ // NWORK_


def _emb_kernel(xt_hbm, tok_hbm, pos_hbm, out_hbm, pos_v, idx_v,
                tok0, tok1, tr0, tr1, gs0, gs1, os0, os1):
    nc = 2
    wid = lax.axis_index("s") * nc + lax.axis_index("c")
    t0 = wid * TPW_

    pltpu.sync_copy(pos_hbm, pos_v)
    pltpu.sync_copy(xt_hbm.at[pl.ds(t0, TPW_)], idx_v)

    toks = (tok0, tok1)
    trs = (tr0, tr1)
    gsems = (gs0, gs1)
    osems = (os0, os1)

    lanes = lax.iota(jnp.int32, 16)
    ti_c = [(q * 16 + lanes) >> 3 for q in range(EMBED_ // 16)]
    r_c = [(q * 16 + lanes) & 7 for q in range(EMBED_ // 16)]

    def start_gather(i, b):
        pltpu.async_copy(tok_hbm.at[idx_v.at[i]], toks[b], gsems[b])

    def wait_gather(b):
        pltpu.make_async_copy(tok_hbm.at[idx_v.at[0]], toks[b], gsems[b]).wait()

    def start_out(t, b):
        s = t // NBBLK_
        tj = lax.rem(t, NBBLK_)
        pltpu.async_copy(trs[b].at[:, :, pl.ds(0, BBLK_)],
                         out_hbm.at[s, slice(None), tj], osems[b])

    def wait_out(b):
        pltpu.make_async_copy(trs[b].at[:, :, pl.ds(0, BBLK_)],
                              out_hbm.at[0, slice(None), 0], osems[b]).wait()

    start_gather(0, 0)

    def super_body(kk, carry):
        for b in (0, 1):
            i = kk * 2 + b
            t = t0 + i
            nb = 1 - b
            wait_gather(b)

            @pl.when(i + 1 < TPW_)
            def _():
                start_gather(i + 1, nb)

            @pl.when(i >= 2)
            def _():
                wait_out(b)

            s = t // NBBLK_
            tok = toks[b]
            tr = trs[b]
            prow = [pos_v[s, pl.ds(q * 16, 16)] for q in range(EMBED_ // 16)]

            @plsc.parallel_loop(0, BBLK_, unroll=4,
                                carry=jnp.broadcast_to(0, (16,)))
            def j_loop(j, jvec):
                for q in range(EMBED_ // 16):
                    vals = tok[j, pl.ds(q * 16, 16)] + prow[q]
                    plsc.store_scatter(tr, [ti_c[q], r_c[q], jvec], vals)
                return jvec + 1

            start_out(t, b)
        return carry

    lax.fori_loop(0, TPW_ // 2, super_body, 0)
    wait_out(0)
    wait_out(1)


def kernel(x, token_table, pos_table):
    batch, seqlen = x.shape
    xt = jnp.transpose(x.astype(jnp.int32), (1, 0)).reshape(NTASK_, BBLK_)

    mesh = plsc.VectorSubcoreMesh(core_axis_name="c", subcore_axis_name="s")
    run = functools.partial(
        pl.kernel,
        mesh=mesh,
        compiler_params=pltpu.CompilerParams(
            use_tc_tiling_on_sc=False, needs_layout_passes=False),
        out_type=jax.ShapeDtypeStruct(
            (MAXLEN_, EMBED_ // 8, NBBLK_, 8, BBLK_), jnp.float32),
        scratch_types=[
            pltpu.VMEM((MAXLEN_, EMBED_), jnp.float32),
            pltpu.VMEM((TPW_, BBLK_), jnp.int32),
            pltpu.VMEM((BBLK_, EMBED_), jnp.float32),
            pltpu.VMEM((BBLK_, EMBED_), jnp.float32),
            pltpu.VMEM((EMBED_ // 8, 8, BBLK_ + 1), jnp.float32),
            pltpu.VMEM((EMBED_ // 8, 8, BBLK_ + 1), jnp.float32),
            pltpu.SemaphoreType.DMA,
            pltpu.SemaphoreType.DMA,
            pltpu.SemaphoreType.DMA,
            pltpu.SemaphoreType.DMA,
        ],
    )(_emb_kernel)
    out5 = run(xt, token_table, pos_table)
    out = jnp.transpose(out5, (0, 1, 3, 2, 4)).reshape(MAXLEN_, EMBED_, BATCH_)
    return jnp.transpose(out, (2, 0, 1))
